# baseline (speedup 1.0000x reference)
import numpy as np
import jax
import jax.numpy as jnp
from jax import lax
from jax.experimental import pallas as pl
from jax.experimental.pallas import tpu as pltpu

N = 8192
NB = 8
K = 5
BNS = float(1.0 / np.sqrt(1.0 + 1e-5))
ROWT = 256
CW = 1280
NEG_BIG = -1e9

_NN = (((1,), (0,)), ((), ()))
_NT = (((1,), (1,)), ((), ()))
_TN = (((0,), (0,)), ((), ()))


def _mm(a, b):
    return lax.dot_general(a, b, _NN, preferred_element_type=jnp.float32)


def _mmt(a, b):
    return lax.dot_general(a, b, _NT, preferred_element_type=jnp.float32)


def _mtn(a, b):
    return lax.dot_general(a, b, _TN, preferred_element_type=jnp.float32)


def _mlp(h, w1_ref, bb1_ref, g1_ref, w2_ref, bb2_ref, g2_ref,
         w3_ref, b3_ref, sw_ref, sb_ref, hw_ref, hb_ref):
    bb1 = bb1_ref[...]
    x1 = jax.nn.relu(_mm(h, w1_ref[...]) + bb1[0:1, :])
    x1 = x1 * (g1_ref[...] * BNS) + bb1[1:2, :]
    bb2 = bb2_ref[...]
    x2 = jax.nn.relu(_mm(x1, w2_ref[...]) + bb2[0:1, :])
    x2 = x2 * (g2_ref[...] * BNS) + bb2[1:2, :]
    feat = _mm(x2, w3_ref[...]) + b3_ref[...]
    s = _mm(feat, sw_ref[...]) + sb_ref[...]
    hm = _mm(feat, hw_ref[...]) + hb_ref[...]
    return feat, s, hm


def _mlp_params(p, i):
    return (
        p[f'b{i}_W1'].T,
        jnp.stack([p[f'b{i}_b1'], p[f'b{i}_be1']]),
        p[f'b{i}_g1'][None, :],
        p[f'b{i}_W2'].T,
        jnp.stack([p[f'b{i}_b2'], p[f'b{i}_be2']]),
        p[f'b{i}_g2'][None, :],
        p[f'b{i}_W3'].T,
        p[f'b{i}_b3'][None, :],
        p[f'b{i}_sW'].T,
        p[f'b{i}_sb'][None, :],
        p[f'b{i}_hW'].T,
        p[f'b{i}_hb'][None, :],
    )


_MLP_SPECS = lambda cin: [
    pl.BlockSpec((cin, 128), lambda i, *_: (0, 0)),
    pl.BlockSpec((2, 128), lambda i, *_: (0, 0)),
    pl.BlockSpec((1, 128), lambda i, *_: (0, 0)),
    pl.BlockSpec((128, 128), lambda i, *_: (0, 0)),
    pl.BlockSpec((2, 128), lambda i, *_: (0, 0)),
    pl.BlockSpec((1, 128), lambda i, *_: (0, 0)),
    pl.BlockSpec((128, 64), lambda i, *_: (0, 0)),
    pl.BlockSpec((1, 64), lambda i, *_: (0, 0)),
    pl.BlockSpec((64, 16), lambda i, *_: (0, 0)),
    pl.BlockSpec((1, 16), lambda i, *_: (0, 0)),
    pl.BlockSpec((64, 64), lambda i, *_: (0, 0)),
    pl.BlockSpec((1, 64), lambda i, *_: (0, 0)),
]


def _a_body(x_ref, b_ref, *refs):
    mlp_refs, out_refs = refs[:12], refs[12:]
    x = x_ref[...]
    b = b_ref[...]
    oneh = (b == lax.broadcasted_iota(jnp.int32, (1, NB), 1)).astype(jnp.float32)
    ones_col = jnp.ones((N, 1), jnp.float32)
    cnt = _mtn(oneh, ones_col)
    ssum = _mtn(oneh, x)
    mean = ssum / jnp.maximum(cnt, 1.0)
    mns, mxs = [], []
    for bb in range(NB):
        m = b == bb
        mns.append(jnp.min(jnp.where(m, x, jnp.inf), axis=0, keepdims=True))
        mxs.append(jnp.max(jnp.where(m, x, -jnp.inf), axis=0, keepdims=True))
    mn = jnp.concatenate(mns, axis=0)
    mx = jnp.concatenate(mxs, axis=0)
    nonempty = cnt > 0.0
    mn = jnp.where(nonempty, mn, 0.0)
    mx = jnp.where(nonempty, mx, 0.0)
    mmm = jnp.concatenate([mean, mn, mx], axis=1)
    h80 = jnp.concatenate([_mm(oneh, mmm), x], axis=1)
    feat, s, hm = _mlp(h80, *mlp_refs)
    out_refs[0][...] = feat
    out_refs[1][...] = s
    out_refs[2][...] = hm


def _a_call(x, bcol, p):
    return pl.pallas_call(
        _a_body,
        out_shape=[
            jax.ShapeDtypeStruct((N, 64), jnp.float32),
            jax.ShapeDtypeStruct((N, 16), jnp.float32),
            jax.ShapeDtypeStruct((N, 64), jnp.float32),
        ],
    )(x, bcol, *_mlp_params(p, 0))


def _grav_core(t, lo8_ref, nch_ref, sr_ref, br_ref, feat_ref, s_ref,
               ball_ref, h_ref, ow_ref, obp_ref, pg_ref):
    l0 = lo8_ref[t]
    nc = nch_ref[t]
    s_r = sr_ref[...]
    sq_r = jnp.sum(s_r * s_r, axis=1, keepdims=True)
    aug_r = jnp.concatenate([s_r * -2.0, jnp.ones((ROWT, 1), jnp.float32)],
                            axis=1)
    br = br_ref[...]
    liota = lax.broadcasted_iota(jnp.int32, (ROWT, CW), 1)

    def chunk_base(j):
        return pl.multiple_of(jnp.minimum(l0 + j * CW, N - CW), 128)

    def chunk_negd(base):
        s_c = s_ref[pl.ds(base, CW), :]
        sq_c = jnp.sum(s_c * s_c, axis=1, keepdims=True)
        aug_c = jnp.concatenate([s_c, sq_c], axis=1)
        d2 = sq_r + _mmt(aug_r, aug_c)
        d2 = jnp.maximum(d2, 0.0)
        bc = ball_ref[:, pl.ds(base, CW)]
        same = br == bc
        return jnp.where(same, -d2, NEG_BIG)

    def phase1(j, carry):
        bval, bidx = carry
        base = chunk_base(j)
        negd = chunk_negd(base)
        gcols = liota + base
        cval = jnp.concatenate([bval, negd], axis=1)
        cidx = jnp.concatenate([bidx, gcols], axis=1)
        nv, ni = [], []
        for _ in range(K):
            m = jnp.max(cval, axis=1, keepdims=True)
            gidx = jnp.min(jnp.where(cval == m, cidx, jnp.int32(1 << 30)),
                           axis=1, keepdims=True)
            cval = jnp.where(cidx == gidx, -jnp.inf, cval)
            nv.append(m)
            ni.append(gidx)
        return (jnp.concatenate(nv, axis=1), jnp.concatenate(ni, axis=1))

    bval0 = jnp.full((ROWT, K), -jnp.inf, jnp.float32)
    bidx0 = jnp.int32(1 << 30) + lax.broadcasted_iota(jnp.int32, (ROWT, K), 1)
    bval, bidx = lax.fori_loop(0, nc, phase1, (bval0, bidx0))

    def phase2(j, carry):
        base = chunk_base(j)
        h_c = h_ref[pl.ds(base, CW), :]
        gcols = liota + base
        gmask = jnp.where(gcols >= l0 + j * CW, gcols, -1)
        out = []
        for k in range(K):
            oneh = (gmask == bidx[:, k:k + 1]).astype(jnp.float32)
            out.append(carry[k] + _mm(oneh, h_c))
        return tuple(out)

    g0 = tuple(jnp.zeros((ROWT, 64), jnp.float32) for _ in range(K))
    gs = lax.fori_loop(0, nc, phase2, g0)

    msum = jnp.zeros((ROWT, 64), jnp.float32)
    mmax = jnp.full((ROWT, 64), -jnp.inf, jnp.float32)
    for k in range(K):
        msg = gs[k] * jnp.exp(10.0 * bval[:, k:k + 1])
        msum = msum + msg
        mmax = jnp.maximum(mmax, msg)
    outcat = jnp.concatenate([feat_ref[...], msum * (1.0 / K), mmax], axis=1)
    obp = obp_ref[...]
    o = _mm(outcat, ow_ref[...]) + obp[0:1, :]
    return o * (pg_ref[...] * BNS) + obp[1:2, :]


_GRAV_SPECS = [
    pl.BlockSpec((ROWT, 16), lambda i, *_: (i, 0)),
    pl.BlockSpec((ROWT, 1), lambda i, *_: (i, 0)),
    pl.BlockSpec((ROWT, 64), lambda i, *_: (i, 0)),
    pl.BlockSpec((N, 16), lambda i, *_: (0, 0)),
    pl.BlockSpec((1, N), lambda i, *_: (0, 0)),
    pl.BlockSpec((N, 64), lambda i, *_: (0, 0)),
    pl.BlockSpec((192, 64), lambda i, *_: (0, 0)),
    pl.BlockSpec((2, 64), lambda i, *_: (0, 0)),
    pl.BlockSpec((1, 64), lambda i, *_: (0, 0)),
]


def _grav_params(p, i):
    return (
        p[f'b{i}_oW'].T,
        jnp.stack([p[f'b{i}_ob'], p[f'b{i}_pb']]),
        p[f'b{i}_pg'][None, :],
    )


def _b_body(lo8_ref, nch_ref, sr_ref, br_ref, feat_ref, s_ref, ball_ref,
            h_ref, ow_ref, obp_ref, pg_ref, *refs):
    mlp_refs, out_refs = refs[:12], refs[12:]
    t = pl.program_id(0)
    o = _grav_core(t, lo8_ref, nch_ref, sr_ref, br_ref, feat_ref, s_ref,
                   ball_ref, h_ref, ow_ref, obp_ref, pg_ref)
    feat, s, hm = _mlp(o, *mlp_refs)
    out_refs[0][...] = o
    out_refs[1][...] = feat
    out_refs[2][...] = s
    out_refs[3][...] = hm


def _b_call(s, bcol, brow, feat, h, p, i, lo8, nch):
    return pl.pallas_call(
        _b_body,
        grid_spec=pltpu.PrefetchScalarGridSpec(
            num_scalar_prefetch=2,
            grid=(N // ROWT,),
            in_specs=_GRAV_SPECS + _MLP_SPECS(64),
            out_specs=[
                pl.BlockSpec((ROWT, 64), lambda i, *_: (i, 0)),
                pl.BlockSpec((ROWT, 64), lambda i, *_: (i, 0)),
                pl.BlockSpec((ROWT, 16), lambda i, *_: (i, 0)),
                pl.BlockSpec((ROWT, 64), lambda i, *_: (i, 0)),
            ],
        ),
        out_shape=[
            jax.ShapeDtypeStruct((N, 64), jnp.float32),
            jax.ShapeDtypeStruct((N, 64), jnp.float32),
            jax.ShapeDtypeStruct((N, 16), jnp.float32),
            jax.ShapeDtypeStruct((N, 64), jnp.float32),
        ],
        compiler_params=pltpu.CompilerParams(
            dimension_semantics=("parallel",)),
    )(lo8, nch, s, bcol, feat, s, brow, h, *_grav_params(p, i),
      *_mlp_params(p, i + 1))


def _c_body(lo8_ref, nch_ref, sr_ref, br_ref, feat_ref, s_ref, ball_ref,
            h_ref, ow_ref, obp_ref, pg_ref, b0_ref, b1_ref, *refs):
    head_refs, out_refs = refs[:11], refs[11:]
    t = pl.program_id(0)
    o = _grav_core(t, lo8_ref, nch_ref, sr_ref, br_ref, feat_ref, s_ref,
                   ball_ref, h_ref, ow_ref, obp_ref, pg_ref)
    h = jnp.concatenate([b0_ref[...], b1_ref[...], o], axis=1)
    (w0, bb0, g0, w1, bb1, g1, w2, bb2, g2, wo, bo) = head_refs
    for w_ref, bb_ref, g_ref in ((w0, bb0, g0), (w1, bb1, g1), (w2, bb2, g2)):
        bb = bb_ref[...]
        h = jax.nn.relu(_mm(h, w_ref[...]) + bb[0:1, :])
        h = h * (g_ref[...] * BNS) + bb[1:2, :]
    out_refs[0][...] = jax.nn.sigmoid(_mm(h, wo[...]) + bo[...])


def _c_call(s, bcol, brow, feat, h, blk0, blk1, p, lo8, nch):
    head_args = []
    head_specs = []
    for j in range(3):
        cin = 192 if j == 0 else 256
        head_args += [p[f'd{j}_W'].T,
                      jnp.stack([p[f'd{j}_b'], p[f'd{j}_be']]),
                      p[f'd{j}_g'][None, :]]
        head_specs += [pl.BlockSpec((cin, 256), lambda i, *_: (0, 0)),
                       pl.BlockSpec((2, 256), lambda i, *_: (0, 0)),
                       pl.BlockSpec((1, 256), lambda i, *_: (0, 0))]
    head_args += [p['out_W'].T, p['out_b'][None, :]]
    head_specs += [pl.BlockSpec((256, 1), lambda i, *_: (0, 0)),
                   pl.BlockSpec((1, 1), lambda i, *_: (0, 0))]
    return pl.pallas_call(
        _c_body,
        grid_spec=pltpu.PrefetchScalarGridSpec(
            num_scalar_prefetch=2,
            grid=(N // ROWT,),
            in_specs=_GRAV_SPECS
            + [pl.BlockSpec((ROWT, 64), lambda i, *_: (i, 0)),
               pl.BlockSpec((ROWT, 64), lambda i, *_: (i, 0))]
            + head_specs,
            out_specs=[pl.BlockSpec((ROWT, 1), lambda i, *_: (i, 0))],
        ),
        out_shape=[jax.ShapeDtypeStruct((N, 1), jnp.float32)],
        compiler_params=pltpu.CompilerParams(
            dimension_semantics=("parallel",)),
    )(lo8, nch, s, bcol, feat, s, brow, h, *_grav_params(p, 2),
      blk0, blk1, *head_args)


def kernel(x, batch, params):
    b32 = batch.astype(jnp.int32)
    bcol = b32.reshape(N, 1)
    brow = b32.reshape(1, N)
    seg_start = jnp.searchsorted(b32, jnp.arange(NB, dtype=jnp.int32),
                                 side='left').astype(jnp.int32)
    seg_end = jnp.searchsorted(b32, jnp.arange(NB, dtype=jnp.int32),
                               side='right').astype(jnp.int32)
    bt0 = b32[::ROWT]
    bt1 = b32[ROWT - 1::ROWT]
    lo8 = (seg_start[bt0] & ~127).astype(jnp.int32)
    nch = ((seg_end[bt1] - lo8 + CW - 1) // CW).astype(jnp.int32)
    feat, s, hm = _a_call(x, bcol, params)
    blk0, feat, s, hm = _b_call(s, bcol, brow, feat, hm, params, 0, lo8, nch)
    blk1, feat, s, hm = _b_call(s, bcol, brow, feat, hm, params, 1, lo8, nch)
    (out,) = _c_call(s, bcol, brow, feat, hm, blk0, blk1, params, lo8, nch)
    return out


# speedup vs baseline: 1.0608x; 1.0608x over previous
import numpy as np
import jax
import jax.numpy as jnp
from jax import lax
from jax.experimental import pallas as pl
from jax.experimental.pallas import tpu as pltpu

N = 8192
NB = 8
K = 5
BNS = float(1.0 / np.sqrt(1.0 + 1e-5))
ROWT = 512
CW = 1280
NEG_BIG = -1e9

_NN = (((1,), (0,)), ((), ()))
_NT = (((1,), (1,)), ((), ()))
_TN = (((0,), (0,)), ((), ()))


def _mm(a, b):
    return lax.dot_general(a, b, _NN, preferred_element_type=jnp.float32)


def _mmt(a, b):
    return lax.dot_general(a, b, _NT, preferred_element_type=jnp.float32)


def _mtn(a, b):
    return lax.dot_general(a, b, _TN, preferred_element_type=jnp.float32)


def _mlp(h, w1_ref, bb1_ref, g1_ref, w2_ref, bb2_ref, g2_ref,
         w3_ref, b3_ref, sw_ref, sb_ref, hw_ref, hb_ref):
    bb1 = bb1_ref[...]
    x1 = jax.nn.relu(_mm(h, w1_ref[...]) + bb1[0:1, :])
    x1 = x1 * (g1_ref[...] * BNS) + bb1[1:2, :]
    bb2 = bb2_ref[...]
    x2 = jax.nn.relu(_mm(x1, w2_ref[...]) + bb2[0:1, :])
    x2 = x2 * (g2_ref[...] * BNS) + bb2[1:2, :]
    feat = _mm(x2, w3_ref[...]) + b3_ref[...]
    s = _mm(feat, sw_ref[...]) + sb_ref[...]
    hm = _mm(feat, hw_ref[...]) + hb_ref[...]
    return feat, s, hm


def _mlp_params(p, i):
    return (
        p[f'b{i}_W1'].T,
        jnp.stack([p[f'b{i}_b1'], p[f'b{i}_be1']]),
        p[f'b{i}_g1'][None, :],
        p[f'b{i}_W2'].T,
        jnp.stack([p[f'b{i}_b2'], p[f'b{i}_be2']]),
        p[f'b{i}_g2'][None, :],
        p[f'b{i}_W3'].T,
        p[f'b{i}_b3'][None, :],
        p[f'b{i}_sW'].T,
        p[f'b{i}_sb'][None, :],
        p[f'b{i}_hW'].T,
        p[f'b{i}_hb'][None, :],
    )


_MLP_SPECS = lambda cin: [
    pl.BlockSpec((cin, 128), lambda i, *_: (0, 0)),
    pl.BlockSpec((2, 128), lambda i, *_: (0, 0)),
    pl.BlockSpec((1, 128), lambda i, *_: (0, 0)),
    pl.BlockSpec((128, 128), lambda i, *_: (0, 0)),
    pl.BlockSpec((2, 128), lambda i, *_: (0, 0)),
    pl.BlockSpec((1, 128), lambda i, *_: (0, 0)),
    pl.BlockSpec((128, 64), lambda i, *_: (0, 0)),
    pl.BlockSpec((1, 64), lambda i, *_: (0, 0)),
    pl.BlockSpec((64, 16), lambda i, *_: (0, 0)),
    pl.BlockSpec((1, 16), lambda i, *_: (0, 0)),
    pl.BlockSpec((64, 64), lambda i, *_: (0, 0)),
    pl.BlockSpec((1, 64), lambda i, *_: (0, 0)),
]


def _a_body(x_ref, b_ref, *refs):
    mlp_refs, out_refs = refs[:12], refs[12:]
    x = x_ref[...]
    b = b_ref[...]
    oneh = (b == lax.broadcasted_iota(jnp.int32, (1, NB), 1)).astype(jnp.float32)
    ones_col = jnp.ones((N, 1), jnp.float32)
    cnt = _mtn(oneh, ones_col)
    ssum = _mtn(oneh, x)
    mean = ssum / jnp.maximum(cnt, 1.0)
    mns, mxs = [], []
    for bb in range(NB):
        m = b == bb
        mns.append(jnp.min(jnp.where(m, x, jnp.inf), axis=0, keepdims=True))
        mxs.append(jnp.max(jnp.where(m, x, -jnp.inf), axis=0, keepdims=True))
    mn = jnp.concatenate(mns, axis=0)
    mx = jnp.concatenate(mxs, axis=0)
    nonempty = cnt > 0.0
    mn = jnp.where(nonempty, mn, 0.0)
    mx = jnp.where(nonempty, mx, 0.0)
    mmm = jnp.concatenate([mean, mn, mx], axis=1)
    h80 = jnp.concatenate([_mm(oneh, mmm), x], axis=1)
    feat, s, hm = _mlp(h80, *mlp_refs)
    out_refs[0][...] = feat
    out_refs[1][...] = s
    out_refs[2][...] = hm


def _a_call(x, bcol, p):
    return pl.pallas_call(
        _a_body,
        out_shape=[
            jax.ShapeDtypeStruct((N, 64), jnp.float32),
            jax.ShapeDtypeStruct((N, 16), jnp.float32),
            jax.ShapeDtypeStruct((N, 64), jnp.float32),
        ],
    )(x, bcol, *_mlp_params(p, 0))


def _grav_core(t, lo8_ref, nch_ref, sr_ref, br_ref, feat_ref, s_ref,
               ball_ref, h_ref, ow_ref, obp_ref, pg_ref):
    l0 = lo8_ref[t]
    nc = nch_ref[t]
    s_r = sr_ref[...]
    sq_r = jnp.sum(s_r * s_r, axis=1, keepdims=True)
    aug_r = jnp.concatenate([s_r * -2.0, jnp.ones((ROWT, 1), jnp.float32)],
                            axis=1)
    br = br_ref[...]
    liota = lax.broadcasted_iota(jnp.int32, (ROWT, CW), 1)

    def chunk_base(j):
        return pl.multiple_of(jnp.minimum(l0 + j * CW, N - CW), 128)

    def chunk_negd(base):
        s_c = s_ref[pl.ds(base, CW), :]
        sq_c = jnp.sum(s_c * s_c, axis=1, keepdims=True)
        aug_c = jnp.concatenate([s_c, sq_c], axis=1)
        d2 = sq_r + _mmt(aug_r, aug_c)
        d2 = jnp.maximum(d2, 0.0)
        bc = ball_ref[:, pl.ds(base, CW)]
        same = br == bc
        return jnp.where(same, -d2, NEG_BIG)

    def phase1(j, carry):
        bval, bidx = carry
        base = chunk_base(j)
        negd = chunk_negd(base)
        gcols = liota + base
        cval = jnp.concatenate([bval, negd], axis=1)
        cidx = jnp.concatenate([bidx, gcols], axis=1)
        nv, ni = [], []
        for _ in range(K):
            m = jnp.max(cval, axis=1, keepdims=True)
            gidx = jnp.min(jnp.where(cval == m, cidx, jnp.int32(1 << 30)),
                           axis=1, keepdims=True)
            cval = jnp.where(cidx == gidx, -jnp.inf, cval)
            nv.append(m)
            ni.append(gidx)
        return (jnp.concatenate(nv, axis=1), jnp.concatenate(ni, axis=1))

    bval0 = jnp.full((ROWT, K), -jnp.inf, jnp.float32)
    bidx0 = jnp.int32(1 << 30) + lax.broadcasted_iota(jnp.int32, (ROWT, K), 1)
    bval, bidx = lax.fori_loop(0, nc, phase1, (bval0, bidx0))

    def phase2(j, carry):
        base = chunk_base(j)
        h_c = h_ref[pl.ds(base, CW), :]
        gcols = liota + base
        gmask = jnp.where(gcols >= l0 + j * CW, gcols, -1)
        out = []
        for k in range(K):
            oneh = (gmask == bidx[:, k:k + 1]).astype(jnp.float32)
            out.append(carry[k] + _mm(oneh, h_c))
        return tuple(out)

    g0 = tuple(jnp.zeros((ROWT, 64), jnp.float32) for _ in range(K))
    gs = lax.fori_loop(0, nc, phase2, g0)

    msum = jnp.zeros((ROWT, 64), jnp.float32)
    mmax = jnp.full((ROWT, 64), -jnp.inf, jnp.float32)
    for k in range(K):
        msg = gs[k] * jnp.exp(10.0 * bval[:, k:k + 1])
        msum = msum + msg
        mmax = jnp.maximum(mmax, msg)
    outcat = jnp.concatenate([feat_ref[...], msum * (1.0 / K), mmax], axis=1)
    obp = obp_ref[...]
    o = _mm(outcat, ow_ref[...]) + obp[0:1, :]
    return o * (pg_ref[...] * BNS) + obp[1:2, :]


_GRAV_SPECS = [
    pl.BlockSpec((ROWT, 16), lambda i, *_: (i, 0)),
    pl.BlockSpec((ROWT, 1), lambda i, *_: (i, 0)),
    pl.BlockSpec((ROWT, 64), lambda i, *_: (i, 0)),
    pl.BlockSpec((N, 16), lambda i, *_: (0, 0)),
    pl.BlockSpec((1, N), lambda i, *_: (0, 0)),
    pl.BlockSpec((N, 64), lambda i, *_: (0, 0)),
    pl.BlockSpec((192, 64), lambda i, *_: (0, 0)),
    pl.BlockSpec((2, 64), lambda i, *_: (0, 0)),
    pl.BlockSpec((1, 64), lambda i, *_: (0, 0)),
]


def _grav_params(p, i):
    return (
        p[f'b{i}_oW'].T,
        jnp.stack([p[f'b{i}_ob'], p[f'b{i}_pb']]),
        p[f'b{i}_pg'][None, :],
    )


def _b_body(lo8_ref, nch_ref, sr_ref, br_ref, feat_ref, s_ref, ball_ref,
            h_ref, ow_ref, obp_ref, pg_ref, *refs):
    mlp_refs, out_refs = refs[:12], refs[12:]
    t = pl.program_id(0)
    o = _grav_core(t, lo8_ref, nch_ref, sr_ref, br_ref, feat_ref, s_ref,
                   ball_ref, h_ref, ow_ref, obp_ref, pg_ref)
    feat, s, hm = _mlp(o, *mlp_refs)
    out_refs[0][...] = o
    out_refs[1][...] = feat
    out_refs[2][...] = s
    out_refs[3][...] = hm


def _b_call(s, bcol, brow, feat, h, p, i, lo8, nch):
    return pl.pallas_call(
        _b_body,
        grid_spec=pltpu.PrefetchScalarGridSpec(
            num_scalar_prefetch=2,
            grid=(N // ROWT,),
            in_specs=_GRAV_SPECS + _MLP_SPECS(64),
            out_specs=[
                pl.BlockSpec((ROWT, 64), lambda i, *_: (i, 0)),
                pl.BlockSpec((ROWT, 64), lambda i, *_: (i, 0)),
                pl.BlockSpec((ROWT, 16), lambda i, *_: (i, 0)),
                pl.BlockSpec((ROWT, 64), lambda i, *_: (i, 0)),
            ],
        ),
        out_shape=[
            jax.ShapeDtypeStruct((N, 64), jnp.float32),
            jax.ShapeDtypeStruct((N, 64), jnp.float32),
            jax.ShapeDtypeStruct((N, 16), jnp.float32),
            jax.ShapeDtypeStruct((N, 64), jnp.float32),
        ],
        compiler_params=pltpu.CompilerParams(
            dimension_semantics=("parallel",)),
    )(lo8, nch, s, bcol, feat, s, brow, h, *_grav_params(p, i),
      *_mlp_params(p, i + 1))


def _c_body(lo8_ref, nch_ref, sr_ref, br_ref, feat_ref, s_ref, ball_ref,
            h_ref, ow_ref, obp_ref, pg_ref, b0_ref, b1_ref, *refs):
    head_refs, out_refs = refs[:11], refs[11:]
    t = pl.program_id(0)
    o = _grav_core(t, lo8_ref, nch_ref, sr_ref, br_ref, feat_ref, s_ref,
                   ball_ref, h_ref, ow_ref, obp_ref, pg_ref)
    h = jnp.concatenate([b0_ref[...], b1_ref[...], o], axis=1)
    (w0, bb0, g0, w1, bb1, g1, w2, bb2, g2, wo, bo) = head_refs
    for w_ref, bb_ref, g_ref in ((w0, bb0, g0), (w1, bb1, g1), (w2, bb2, g2)):
        bb = bb_ref[...]
        h = jax.nn.relu(_mm(h, w_ref[...]) + bb[0:1, :])
        h = h * (g_ref[...] * BNS) + bb[1:2, :]
    out_refs[0][...] = jax.nn.sigmoid(_mm(h, wo[...]) + bo[...])


def _c_call(s, bcol, brow, feat, h, blk0, blk1, p, lo8, nch):
    head_args = []
    head_specs = []
    for j in range(3):
        cin = 192 if j == 0 else 256
        head_args += [p[f'd{j}_W'].T,
                      jnp.stack([p[f'd{j}_b'], p[f'd{j}_be']]),
                      p[f'd{j}_g'][None, :]]
        head_specs += [pl.BlockSpec((cin, 256), lambda i, *_: (0, 0)),
                       pl.BlockSpec((2, 256), lambda i, *_: (0, 0)),
                       pl.BlockSpec((1, 256), lambda i, *_: (0, 0))]
    head_args += [p['out_W'].T, p['out_b'][None, :]]
    head_specs += [pl.BlockSpec((256, 1), lambda i, *_: (0, 0)),
                   pl.BlockSpec((1, 1), lambda i, *_: (0, 0))]
    return pl.pallas_call(
        _c_body,
        grid_spec=pltpu.PrefetchScalarGridSpec(
            num_scalar_prefetch=2,
            grid=(N // ROWT,),
            in_specs=_GRAV_SPECS
            + [pl.BlockSpec((ROWT, 64), lambda i, *_: (i, 0)),
               pl.BlockSpec((ROWT, 64), lambda i, *_: (i, 0))]
            + head_specs,
            out_specs=[pl.BlockSpec((ROWT, 1), lambda i, *_: (i, 0))],
        ),
        out_shape=[jax.ShapeDtypeStruct((N, 1), jnp.float32)],
        compiler_params=pltpu.CompilerParams(
            dimension_semantics=("parallel",)),
    )(lo8, nch, s, bcol, feat, s, brow, h, *_grav_params(p, 2),
      blk0, blk1, *head_args)


def kernel(x, batch, params):
    b32 = batch.astype(jnp.int32)
    bcol = b32.reshape(N, 1)
    brow = b32.reshape(1, N)
    seg_start = jnp.searchsorted(b32, jnp.arange(NB, dtype=jnp.int32),
                                 side='left').astype(jnp.int32)
    seg_end = jnp.searchsorted(b32, jnp.arange(NB, dtype=jnp.int32),
                               side='right').astype(jnp.int32)
    bt0 = b32[::ROWT]
    bt1 = b32[ROWT - 1::ROWT]
    lo8 = (seg_start[bt0] & ~127).astype(jnp.int32)
    nch = ((seg_end[bt1] - lo8 + CW - 1) // CW).astype(jnp.int32)
    feat, s, hm = _a_call(x, bcol, params)
    blk0, feat, s, hm = _b_call(s, bcol, brow, feat, hm, params, 0, lo8, nch)
    blk1, feat, s, hm = _b_call(s, bcol, brow, feat, hm, params, 1, lo8, nch)
    (out,) = _c_call(s, bcol, brow, feat, hm, blk0, blk1, params, lo8, nch)
    return out
